# baseline (speedup 1.0000x reference)
import functools

import jax
import jax.numpy as jnp
from jax import lax
from jax.experimental import pallas as pl
from jax.experimental.pallas import tpu as pltpu
from jax.experimental.pallas import tpu_sc as plsc

_NC = 2
_NS = 16
_NW = _NC * _NS
_L = 16


def _sc_gather(batch, in_cols, out_cols, cmap_pad, chunk_rows):
    rows_per_w = batch // _NW
    n_chunks = rows_per_w // chunk_rows
    in_chunk = chunk_rows * in_cols
    out_chunk = chunk_rows * out_cols
    n_vecs = out_chunk // _L

    mesh = plsc.VectorSubcoreMesh(core_axis_name="c", subcore_axis_name="s")

    @functools.partial(
        pl.kernel,
        mesh=mesh,
        out_type=jax.ShapeDtypeStruct((batch * out_cols,), jnp.float32),
        scratch_types=[
            pltpu.VMEM((cmap_pad,), jnp.int32),
            pltpu.VMEM((out_chunk,), jnp.int32),
            pltpu.VMEM((in_chunk,), jnp.float32),
            pltpu.VMEM((out_chunk,), jnp.float32),
        ],
        compiler_params=pltpu.CompilerParams(needs_layout_passes=False),
    )
    def k(in_hbm, cmap_hbm, out_hbm, cmap_v, idx_v, in_v, out_v):
        wid = lax.axis_index("s") * _NC + lax.axis_index("c")
        in_base = wid * (rows_per_w * in_cols)
        out_base = wid * (rows_per_w * out_cols)

        pltpu.sync_copy(cmap_hbm, cmap_v)

        @plsc.parallel_loop(0, out_chunk, step=_L, unroll=8)
        def _(o):
            ov = o + lax.iota(jnp.int32, _L)
            r = ov // out_cols
            kcol = ov - r * out_cols
            cin = plsc.load_gather(cmap_v, [kcol])
            idx_v[pl.ds(o, _L)] = r * in_cols + cin

        for c in range(n_chunks):
            pltpu.sync_copy(
                in_hbm.at[pl.ds(in_base + c * in_chunk, in_chunk)], in_v
            )

            @plsc.parallel_loop(0, out_chunk, step=_L, unroll=8)
            def _(o):
                iv = idx_v[pl.ds(o, _L)]
                out_v[pl.ds(o, _L)] = plsc.load_gather(in_v, [iv])

            pltpu.sync_copy(
                out_v, out_hbm.at[pl.ds(out_base + c * out_chunk, out_chunk)]
            )

    return k


def kernel(joints, joint_maps):
    batch, n_joints, n_coord = joints.shape
    n_map = joint_maps.shape[0]
    in_cols = n_joints * n_coord
    out_cols = n_map * n_coord

    jm = joint_maps.astype(jnp.int32) * n_coord
    cmap = (jm[:, None] + jnp.arange(n_coord, dtype=jnp.int32)).reshape(-1)
    cmap_pad = (out_cols + 7) // 8 * 8
    cmap = jnp.pad(cmap, (0, cmap_pad - out_cols))

    sc = _sc_gather(batch, in_cols, out_cols, cmap_pad, chunk_rows=128)
    out_flat = sc(joints.reshape(-1), cmap)
    return out_flat.reshape(batch, n_map, n_coord)
